# Initial kernel scaffold; baseline (speedup 1.0000x reference)
import jax
import jax.numpy as jnp
from jax.experimental import pallas as pl


def kernel(x, W1, b1, W2, b2, Wr1, br1, Wr2, br2):
    raise NotImplementedError("write your pallas kernel here")


# speedup vs baseline: 1.1593x; 1.1593x over previous
import functools

import jax
import jax.numpy as jnp
from jax import lax
from jax.experimental import pallas as pl
from jax.experimental.pallas import tpu as pltpu
from jax.experimental.pallas import tpu_sc as plsc

N, D, H, O, E, RH = 8192, 768, 768, 768, 8, 256
TB = 1024
NB_R = N // TB
B = 256
NBLK = N // B + E
P = NBLK * B
NC, NS = 2, 16
NW = NC * NS
CH = N // NW
IC = 128

_HI = lax.Precision.HIGHEST


def _router_body(x_ref, wr1_ref, br1_ref, wr2_ref, br2_ref,
                 chosen_ref, ranks_ref, counts_ref, carry_ref):
    t = pl.program_id(0)

    @pl.when(t == 0)
    def _():
        carry_ref[...] = jnp.zeros_like(carry_ref)

    h = jnp.dot(x_ref[...], wr1_ref[...]) + br1_ref[...]
    logits = jnp.dot(h, wr2_ref[...]) + br2_ref[...]
    mx = jnp.max(logits, axis=-1, keepdims=True)
    eidx = lax.broadcasted_iota(jnp.int32, (TB, E), 1)
    chosen = jnp.min(jnp.where(logits >= mx, eidx, E), axis=-1)
    onehot = (eidx == chosen[:, None])
    ri = lax.broadcasted_iota(jnp.int32, (TB, TB), 0)
    ci = lax.broadcasted_iota(jnp.int32, (TB, TB), 1)
    tril = (ci < ri).astype(jnp.float32)
    local = jnp.dot(tril, onehot.astype(jnp.float32), precision=_HI)
    ranks = local + carry_ref[...]
    rank_tok = jnp.sum(jnp.where(onehot, ranks, 0.0), axis=-1)
    chosen_ref[0, 0, :] = chosen
    ranks_ref[0, 0, :] = rank_tok.astype(jnp.int32)
    new_carry = carry_ref[...] + jnp.sum(onehot.astype(jnp.float32), axis=0,
                                         keepdims=True)
    counts_ref[...] = new_carry.astype(jnp.int32)
    carry_ref[...] = new_carry


def _router(x, Wr1, br1, Wr2, br2):
    return pl.pallas_call(
        _router_body,
        grid=(NB_R,),
        in_specs=[
            pl.BlockSpec((TB, D), lambda i: (i, 0)),
            pl.BlockSpec((D, RH), lambda i: (0, 0)),
            pl.BlockSpec((1, RH), lambda i: (0, 0)),
            pl.BlockSpec((RH, E), lambda i: (0, 0)),
            pl.BlockSpec((1, E), lambda i: (0, 0)),
        ],
        out_specs=[
            pl.BlockSpec((1, 1, TB), lambda i: (i, 0, 0)),
            pl.BlockSpec((1, 1, TB), lambda i: (i, 0, 0)),
            pl.BlockSpec((1, E), lambda i: (0, 0)),
        ],
        out_shape=[
            jax.ShapeDtypeStruct((NB_R, 1, TB), jnp.int32),
            jax.ShapeDtypeStruct((NB_R, 1, TB), jnp.int32),
            jax.ShapeDtypeStruct((1, E), jnp.int32),
        ],
        scratch_shapes=[pltpu.VMEM((1, E), jnp.float32)],
    )(x, Wr1, br1[None, :], Wr2, br2[None, :])


def _fold_body(w1_ref, b1_ref, w2_ref, b2_ref, m_ref, c_ref):
    w2 = w2_ref[0]
    m_ref[0] = jnp.dot(w1_ref[0], w2, precision=_HI)
    c_ref[0] = jnp.dot(b1_ref[0], w2, precision=_HI) + b2_ref[0]


def _fold(W1, b1, W2, b2):
    return pl.pallas_call(
        _fold_body,
        grid=(E,),
        in_specs=[
            pl.BlockSpec((1, D, H), lambda e: (e, 0, 0)),
            pl.BlockSpec((1, 1, H), lambda e: (e, 0, 0)),
            pl.BlockSpec((1, H, O), lambda e: (e, 0, 0)),
            pl.BlockSpec((1, 1, O), lambda e: (e, 0, 0)),
        ],
        out_specs=[
            pl.BlockSpec((1, D, O), lambda e: (e, 0, 0)),
            pl.BlockSpec((1, 1, O), lambda e: (e, 0, 0)),
        ],
        out_shape=[
            jax.ShapeDtypeStruct((E, D, O), jnp.float32),
            jax.ShapeDtypeStruct((E, 1, O), jnp.float32),
        ],
    )(W1, b1[:, None, :], W2, b2[:, None, :])


def _dest_body(padoff_ref, chosen_ref, ranks_ref, dest_ref):
    ch = chosen_ref[...]
    off = jnp.zeros_like(ch)
    for e in range(E):
        off = jnp.where(ch == e, padoff_ref[e], off)
    dest_ref[...] = off + ranks_ref[...]


def _dest(pad_off, chosen3, ranks3):
    spec = pltpu.PrefetchScalarGridSpec(
        num_scalar_prefetch=1,
        grid=(1,),
        in_specs=[
            pl.BlockSpec((NB_R, 1, TB), lambda i, po: (0, 0, 0)),
            pl.BlockSpec((NB_R, 1, TB), lambda i, po: (0, 0, 0)),
        ],
        out_specs=pl.BlockSpec((NB_R, 1, TB), lambda i, po: (0, 0, 0)),
    )
    return pl.pallas_call(
        _dest_body,
        grid_spec=spec,
        out_shape=jax.ShapeDtypeStruct((NB_R, 1, TB), jnp.int32),
    )(pad_off, chosen3, ranks3)


def _sc_scatter(x, dest):
    mesh = plsc.VectorSubcoreMesh(core_axis_name="c", subcore_axis_name="s")

    @functools.partial(
        pl.kernel, mesh=mesh,
        out_type=jax.ShapeDtypeStruct((P, D), jnp.float32),
        scratch_types=[
            pltpu.VMEM((CH // IC, IC), jnp.int32),
            pltpu.VMEM((IC, D), jnp.float32),
            pltpu.SemaphoreType.DMA,
        ],
    )
    def k(x_hbm, dest_hbm, xs_hbm, dest_v, rows_v, sem):
        wid = lax.axis_index("s") * NC + lax.axis_index("c")
        base = wid * CH
        pltpu.sync_copy(dest_hbm.at[wid], dest_v)
        for kk in range(CH // IC):
            pltpu.sync_copy(x_hbm.at[pl.ds(base + kk * IC, IC)], rows_v)
            pltpu.async_copy(rows_v, xs_hbm.at[dest_v.at[kk]], sem).wait()

    return k(x, dest)


def _grouped_body(emap_ref, act_ref, x_ref, m_ref, c_ref, o_ref):
    b = pl.program_id(0)

    @pl.when(act_ref[b] != 0)
    def _():
        o_ref[...] = (jnp.dot(x_ref[...], m_ref[0], precision=_HI)
                      + c_ref[0])


def _grouped(emap, act, xs, M, c):
    spec = pltpu.PrefetchScalarGridSpec(
        num_scalar_prefetch=2,
        grid=(NBLK,),
        in_specs=[
            pl.BlockSpec((B, D), lambda b, em, ac: (b, 0)),
            pl.BlockSpec((1, D, O), lambda b, em, ac: (em[b], 0, 0)),
            pl.BlockSpec((1, 1, O), lambda b, em, ac: (em[b], 0, 0)),
        ],
        out_specs=pl.BlockSpec((B, O), lambda b, em, ac: (b, 0)),
    )
    return pl.pallas_call(
        _grouped_body,
        grid_spec=spec,
        out_shape=jax.ShapeDtypeStruct((P, O), jnp.float32),
    )(emap, act, xs, M, c)


def _sc_gather(os_sorted, dest):
    mesh = plsc.VectorSubcoreMesh(core_axis_name="c", subcore_axis_name="s")

    @functools.partial(
        pl.kernel, mesh=mesh,
        out_type=jax.ShapeDtypeStruct((N, O), jnp.float32),
        scratch_types=[
            pltpu.VMEM((CH // IC, IC), jnp.int32),
            pltpu.VMEM((IC, O), jnp.float32),
            pltpu.SemaphoreType.DMA,
        ],
    )
    def k(os_hbm, dest_hbm, out_hbm, dest_v, rows_v, sem):
        wid = lax.axis_index("s") * NC + lax.axis_index("c")
        base = wid * CH
        pltpu.sync_copy(dest_hbm.at[wid], dest_v)
        for kk in range(CH // IC):
            pltpu.async_copy(os_hbm.at[dest_v.at[kk]], rows_v, sem).wait()
            pltpu.sync_copy(rows_v, out_hbm.at[pl.ds(base + kk * IC, IC)])

    return k(os_sorted, dest)


def kernel(x, W1, b1, W2, b2, Wr1, br1, Wr2, br2):
    chosen3, ranks3, counts2 = _router(x, Wr1, br1, Wr2, br2)
    counts = counts2[0]

    M, c = _fold(W1, b1, W2, b2)

    padded = ((counts + B - 1) // B) * B
    pad_off = (jnp.cumsum(padded) - padded).astype(jnp.int32)
    starts = jnp.arange(NBLK, dtype=jnp.int32) * B
    ends = pad_off + padded
    emap_raw = jnp.sum((starts[:, None] >= ends[None, :]).astype(jnp.int32), axis=1)
    emap = jnp.minimum(emap_raw, E - 1).astype(jnp.int32)
    act = ((emap_raw < E) & (starts - pad_off[emap] < counts[emap])).astype(jnp.int32)

    dest3 = _dest(pad_off, chosen3, ranks3)
    dest = dest3.reshape(NW, CH // IC, IC)
    xs = _sc_scatter(x, dest)
    os_sorted = _grouped(emap, act, xs, M, c)
    return _sc_gather(os_sorted, dest)


# speedup vs baseline: 1.8112x; 1.5623x over previous
import functools

import jax
import jax.numpy as jnp
from jax import lax
from jax.experimental import pallas as pl
from jax.experimental.pallas import tpu as pltpu
from jax.experimental.pallas import tpu_sc as plsc

N, D, H, O, E, RH = 8192, 768, 768, 768, 8, 256
TB = 1024
NB_R = N // TB
B = 256
NBLK = N // B + E
P = NBLK * B
NC, NS = 2, 16
NW = NC * NS
CH = N // NW
IC = 128

_HI = lax.Precision.HIGHEST


def _router_body(x_ref, wr1_ref, br1_ref, wr2_ref, br2_ref,
                 chosen_ref, ranks_ref, counts_ref, carry_ref):
    t = pl.program_id(0)

    @pl.when(t == 0)
    def _():
        carry_ref[...] = jnp.zeros_like(carry_ref)

    h = jnp.dot(x_ref[...], wr1_ref[...]) + br1_ref[...]
    logits = jnp.dot(h, wr2_ref[...]) + br2_ref[...]
    mx = jnp.max(logits, axis=-1, keepdims=True)
    eidx = lax.broadcasted_iota(jnp.int32, (TB, E), 1)
    chosen = jnp.min(jnp.where(logits >= mx, eidx, E), axis=-1)
    onehot = (eidx == chosen[:, None])
    ohf = onehot.astype(jnp.float32)
    ri = lax.broadcasted_iota(jnp.int32, (128, 128), 0)
    ci = lax.broadcasted_iota(jnp.int32, (128, 128), 1)
    tril = (ci < ri).astype(jnp.float32)
    carry = carry_ref[...]
    parts = []
    for g in range(TB // 128):
        blk = ohf[g * 128:(g + 1) * 128, :]
        local = jnp.dot(tril, blk)
        parts.append(local + carry)
        carry = carry + local[127:128, :] + blk[127:128, :]
    ranks = jnp.concatenate(parts, axis=0)
    rank_tok = jnp.sum(jnp.where(onehot, ranks, 0.0), axis=-1)
    chosen_ref[0, 0, :] = chosen
    ranks_ref[0, 0, :] = rank_tok.astype(jnp.int32)
    counts_ref[...] = carry.astype(jnp.int32)
    carry_ref[...] = carry


def _router(x, Wr1, br1, Wr2, br2):
    return pl.pallas_call(
        _router_body,
        grid=(NB_R,),
        in_specs=[
            pl.BlockSpec((TB, D), lambda i: (i, 0)),
            pl.BlockSpec((D, RH), lambda i: (0, 0)),
            pl.BlockSpec((1, RH), lambda i: (0, 0)),
            pl.BlockSpec((RH, E), lambda i: (0, 0)),
            pl.BlockSpec((1, E), lambda i: (0, 0)),
        ],
        out_specs=[
            pl.BlockSpec((1, 1, TB), lambda i: (i, 0, 0)),
            pl.BlockSpec((1, 1, TB), lambda i: (i, 0, 0)),
            pl.BlockSpec((1, E), lambda i: (0, 0)),
        ],
        out_shape=[
            jax.ShapeDtypeStruct((NB_R, 1, TB), jnp.int32),
            jax.ShapeDtypeStruct((NB_R, 1, TB), jnp.int32),
            jax.ShapeDtypeStruct((1, E), jnp.int32),
        ],
        scratch_shapes=[pltpu.VMEM((1, E), jnp.float32)],
    )(x, Wr1, br1[None, :], Wr2, br2[None, :])


def _fold_body(w1_ref, b1_ref, w2_ref, b2_ref, m_ref, c_ref):
    w2 = w2_ref[0]
    m_ref[0] = jnp.dot(w1_ref[0], w2)
    c_ref[0] = jnp.dot(b1_ref[0], w2) + b2_ref[0]


def _fold(W1, b1, W2, b2):
    return pl.pallas_call(
        _fold_body,
        grid=(E,),
        in_specs=[
            pl.BlockSpec((1, D, H), lambda e: (e, 0, 0)),
            pl.BlockSpec((1, 1, H), lambda e: (e, 0, 0)),
            pl.BlockSpec((1, H, O), lambda e: (e, 0, 0)),
            pl.BlockSpec((1, 1, O), lambda e: (e, 0, 0)),
        ],
        out_specs=[
            pl.BlockSpec((1, D, O), lambda e: (e, 0, 0)),
            pl.BlockSpec((1, 1, O), lambda e: (e, 0, 0)),
        ],
        out_shape=[
            jax.ShapeDtypeStruct((E, D, O), jnp.float32),
            jax.ShapeDtypeStruct((E, 1, O), jnp.float32),
        ],
    )(W1, b1[:, None, :], W2, b2[:, None, :])


def _dest_body(padoff_ref, chosen_ref, ranks_ref, dest_ref):
    ch = chosen_ref[...]
    off = jnp.zeros_like(ch)
    for e in range(E):
        off = jnp.where(ch == e, padoff_ref[e], off)
    dest_ref[...] = off + ranks_ref[...]


def _dest(pad_off, chosen3, ranks3):
    spec = pltpu.PrefetchScalarGridSpec(
        num_scalar_prefetch=1,
        grid=(1,),
        in_specs=[
            pl.BlockSpec((NB_R, 1, TB), lambda i, po: (0, 0, 0)),
            pl.BlockSpec((NB_R, 1, TB), lambda i, po: (0, 0, 0)),
        ],
        out_specs=pl.BlockSpec((NB_R, 1, TB), lambda i, po: (0, 0, 0)),
    )
    return pl.pallas_call(
        _dest_body,
        grid_spec=spec,
        out_shape=jax.ShapeDtypeStruct((NB_R, 1, TB), jnp.int32),
    )(pad_off, chosen3, ranks3)


def _sc_scatter(x, dest):
    mesh = plsc.VectorSubcoreMesh(core_axis_name="c", subcore_axis_name="s")

    @functools.partial(
        pl.kernel, mesh=mesh,
        out_type=jax.ShapeDtypeStruct((P, D), jnp.float32),
        scratch_types=[
            pltpu.VMEM((CH // IC, IC), jnp.int32),
            pltpu.VMEM((IC, D), jnp.float32),
            pltpu.SemaphoreType.DMA,
        ],
    )
    def k(x_hbm, dest_hbm, xs_hbm, dest_v, rows_v, sem):
        wid = lax.axis_index("s") * NC + lax.axis_index("c")
        base = wid * CH
        pltpu.sync_copy(dest_hbm.at[wid], dest_v)
        for kk in range(CH // IC):
            pltpu.sync_copy(x_hbm.at[pl.ds(base + kk * IC, IC)], rows_v)
            pltpu.async_copy(rows_v, xs_hbm.at[dest_v.at[kk]], sem).wait()

    return k(x, dest)


def _grouped_body(emap_ref, act_ref, x_ref, m_ref, c_ref, o_ref):
    b = pl.program_id(0)

    @pl.when(act_ref[b] != 0)
    def _():
        o_ref[...] = jnp.dot(x_ref[...], m_ref[0]) + c_ref[0]


def _grouped(emap, act, xs, M, c):
    spec = pltpu.PrefetchScalarGridSpec(
        num_scalar_prefetch=2,
        grid=(NBLK,),
        in_specs=[
            pl.BlockSpec((B, D), lambda b, em, ac: (b, 0)),
            pl.BlockSpec((1, D, O), lambda b, em, ac: (em[b], 0, 0)),
            pl.BlockSpec((1, 1, O), lambda b, em, ac: (em[b], 0, 0)),
        ],
        out_specs=pl.BlockSpec((B, O), lambda b, em, ac: (b, 0)),
    )
    return pl.pallas_call(
        _grouped_body,
        grid_spec=spec,
        out_shape=jax.ShapeDtypeStruct((P, O), jnp.float32),
    )(emap, act, xs, M, c)


def _sc_gather(os_sorted, dest):
    mesh = plsc.VectorSubcoreMesh(core_axis_name="c", subcore_axis_name="s")

    @functools.partial(
        pl.kernel, mesh=mesh,
        out_type=jax.ShapeDtypeStruct((N, O), jnp.float32),
        scratch_types=[
            pltpu.VMEM((CH // IC, IC), jnp.int32),
            pltpu.VMEM((IC, O), jnp.float32),
            pltpu.SemaphoreType.DMA,
        ],
    )
    def k(os_hbm, dest_hbm, out_hbm, dest_v, rows_v, sem):
        wid = lax.axis_index("s") * NC + lax.axis_index("c")
        base = wid * CH
        pltpu.sync_copy(dest_hbm.at[wid], dest_v)
        for kk in range(CH // IC):
            pltpu.async_copy(os_hbm.at[dest_v.at[kk]], rows_v, sem).wait()
            pltpu.sync_copy(rows_v, out_hbm.at[pl.ds(base + kk * IC, IC)])

    return k(os_sorted, dest)


def kernel(x, W1, b1, W2, b2, Wr1, br1, Wr2, br2):
    chosen3, ranks3, counts2 = _router(x, Wr1, br1, Wr2, br2)
    counts = counts2[0]

    M, c = _fold(W1, b1, W2, b2)

    padded = ((counts + B - 1) // B) * B
    pad_off = (jnp.cumsum(padded) - padded).astype(jnp.int32)
    starts = jnp.arange(NBLK, dtype=jnp.int32) * B
    ends = pad_off + padded
    emap_raw = jnp.sum((starts[:, None] >= ends[None, :]).astype(jnp.int32), axis=1)
    emap = jnp.minimum(emap_raw, E - 1).astype(jnp.int32)
    act = ((emap_raw < E) & (starts - pad_off[emap] < counts[emap])).astype(jnp.int32)

    dest3 = _dest(pad_off, chosen3, ranks3)
    dest = dest3.reshape(NW, CH // IC, IC)
    xs = _sc_scatter(x, dest)
    os_sorted = _grouped(emap, act, xs, M, c)
    return _sc_gather(os_sorted, dest)


# speedup vs baseline: 1.8952x; 1.0464x over previous
import functools

import jax
import jax.numpy as jnp
from jax import lax
from jax.experimental import pallas as pl
from jax.experimental.pallas import tpu as pltpu
from jax.experimental.pallas import tpu_sc as plsc

N, D, H, O, E, RH = 8192, 768, 768, 768, 8, 256
TB = 1024
NB_R = N // TB
B = 256
NBLK = N // B + E
P = NBLK * B
NC, NS = 2, 16
NW = NC * NS
CH = N // NW
IC = 64
NCHK = CH // IC

_HI = lax.Precision.HIGHEST


def _router_body(x_ref, wr1_ref, br1_ref, wr2_ref, br2_ref,
                 chosen_ref, ranks_ref, counts_ref, carry_ref):
    t = pl.program_id(0)

    @pl.when(t == 0)
    def _():
        carry_ref[...] = jnp.zeros_like(carry_ref)

    h = jnp.dot(x_ref[...], wr1_ref[...]) + br1_ref[...]
    logits = jnp.dot(h, wr2_ref[...]) + br2_ref[...]
    mx = jnp.max(logits, axis=-1, keepdims=True)
    eidx = lax.broadcasted_iota(jnp.int32, (TB, E), 1)
    chosen = jnp.min(jnp.where(logits >= mx, eidx, E), axis=-1)
    onehot = (eidx == chosen[:, None])
    ohf = onehot.astype(jnp.float32)
    ri = lax.broadcasted_iota(jnp.int32, (128, 128), 0)
    ci = lax.broadcasted_iota(jnp.int32, (128, 128), 1)
    tril = (ci < ri).astype(jnp.float32)
    carry = carry_ref[...]
    parts = []
    for g in range(TB // 128):
        blk = ohf[g * 128:(g + 1) * 128, :]
        local = jnp.dot(tril, blk)
        parts.append(local + carry)
        carry = carry + local[127:128, :] + blk[127:128, :]
    ranks = jnp.concatenate(parts, axis=0)
    rank_tok = jnp.sum(jnp.where(onehot, ranks, 0.0), axis=-1)
    chosen_ref[0, 0, :] = chosen
    ranks_ref[0, 0, :] = rank_tok.astype(jnp.int32)
    counts_ref[...] = carry.astype(jnp.int32)
    carry_ref[...] = carry


def _router(x, Wr1, br1, Wr2, br2):
    return pl.pallas_call(
        _router_body,
        grid=(NB_R,),
        in_specs=[
            pl.BlockSpec((TB, D), lambda i: (i, 0)),
            pl.BlockSpec((D, RH), lambda i: (0, 0)),
            pl.BlockSpec((1, RH), lambda i: (0, 0)),
            pl.BlockSpec((RH, E), lambda i: (0, 0)),
            pl.BlockSpec((1, E), lambda i: (0, 0)),
        ],
        out_specs=[
            pl.BlockSpec((1, 1, TB), lambda i: (i, 0, 0)),
            pl.BlockSpec((1, 1, TB), lambda i: (i, 0, 0)),
            pl.BlockSpec((1, E), lambda i: (0, 0)),
        ],
        out_shape=[
            jax.ShapeDtypeStruct((NB_R, 1, TB), jnp.int32),
            jax.ShapeDtypeStruct((NB_R, 1, TB), jnp.int32),
            jax.ShapeDtypeStruct((1, E), jnp.int32),
        ],
        scratch_shapes=[pltpu.VMEM((1, E), jnp.float32)],
    )(x, Wr1, br1[None, :], Wr2, br2[None, :])


def _dest_body(padoff_ref, chosen_ref, ranks_ref, dest_ref):
    ch = chosen_ref[...]
    off = jnp.zeros_like(ch)
    for e in range(E):
        off = jnp.where(ch == e, padoff_ref[e], off)
    dest_ref[...] = off + ranks_ref[...]


def _dest(pad_off, chosen3, ranks3):
    spec = pltpu.PrefetchScalarGridSpec(
        num_scalar_prefetch=1,
        grid=(1,),
        in_specs=[
            pl.BlockSpec((NB_R, 1, TB), lambda i, po: (0, 0, 0)),
            pl.BlockSpec((NB_R, 1, TB), lambda i, po: (0, 0, 0)),
        ],
        out_specs=pl.BlockSpec((NB_R, 1, TB), lambda i, po: (0, 0, 0)),
    )
    return pl.pallas_call(
        _dest_body,
        grid_spec=spec,
        out_shape=jax.ShapeDtypeStruct((NB_R, 1, TB), jnp.int32),
    )(pad_off, chosen3, ranks3)


def _sc_scatter(x, dest):
    mesh = plsc.VectorSubcoreMesh(core_axis_name="c", subcore_axis_name="s")

    @functools.partial(
        pl.kernel, mesh=mesh,
        out_type=jax.ShapeDtypeStruct((P, D), jnp.float32),
        scratch_types=[
            pltpu.VMEM((NCHK, IC), jnp.int32),
            pltpu.VMEM((IC, D), jnp.float32),
            pltpu.VMEM((IC, D), jnp.float32),
            pltpu.SemaphoreType.DMA,
            pltpu.SemaphoreType.DMA,
            pltpu.SemaphoreType.DMA,
        ],
    )
    def k(x_hbm, dest_hbm, xs_hbm, dest_v, rows0, rows1, lsem0, lsem1, ssem):
        wid = lax.axis_index("s") * NC + lax.axis_index("c")
        base = wid * CH
        rows = [rows0, rows1]
        lsem = [lsem0, lsem1]
        pltpu.sync_copy(dest_hbm.at[wid], dest_v)
        ld = [None, None]
        ld[0] = pltpu.async_copy(x_hbm.at[pl.ds(base, IC)], rows0, lsem0)
        for kk in range(NCHK):
            nxt = kk + 1
            if nxt < NCHK:
                ld[nxt % 2] = pltpu.async_copy(
                    x_hbm.at[pl.ds(base + nxt * IC, IC)], rows[nxt % 2],
                    lsem[nxt % 2])
            ld[kk % 2].wait()
            pltpu.async_copy(rows[kk % 2], xs_hbm.at[dest_v.at[kk]],
                             ssem).wait()

    return k(x, dest)


def _grouped_body(emap_ref, act_ref, first_ref, x_ref, w1_ref, b1_ref,
                  w2_ref, b2_ref, o_ref, m_v, c_v):
    b = pl.program_id(0)

    @pl.when((first_ref[b] != 0) & (act_ref[b] != 0))
    def _():
        w2 = w2_ref[0]
        m_v[...] = jnp.dot(w1_ref[0], w2)
        c_v[...] = jnp.dot(b1_ref[0], w2) + b2_ref[0]

    @pl.when(act_ref[b] != 0)
    def _():
        o_ref[...] = jnp.dot(x_ref[...], m_v[...]) + c_v[...]


def _grouped(emap, act, first, xs, W1, b1, W2, b2):
    spec = pltpu.PrefetchScalarGridSpec(
        num_scalar_prefetch=3,
        grid=(NBLK,),
        in_specs=[
            pl.BlockSpec((B, D), lambda b, em, ac, fi: (b, 0)),
            pl.BlockSpec((1, D, H), lambda b, em, ac, fi: (em[b], 0, 0)),
            pl.BlockSpec((1, 1, H), lambda b, em, ac, fi: (em[b], 0, 0)),
            pl.BlockSpec((1, H, O), lambda b, em, ac, fi: (em[b], 0, 0)),
            pl.BlockSpec((1, 1, O), lambda b, em, ac, fi: (em[b], 0, 0)),
        ],
        out_specs=pl.BlockSpec((B, O), lambda b, em, ac, fi: (b, 0)),
        scratch_shapes=[
            pltpu.VMEM((D, O), jnp.float32),
            pltpu.VMEM((1, O), jnp.float32),
        ],
    )
    return pl.pallas_call(
        _grouped_body,
        grid_spec=spec,
        out_shape=jax.ShapeDtypeStruct((P, O), jnp.float32),
    )(emap, act, first, xs, W1, b1[:, None, :], W2, b2[:, None, :])


def _sc_gather(os_sorted, dest):
    mesh = plsc.VectorSubcoreMesh(core_axis_name="c", subcore_axis_name="s")

    @functools.partial(
        pl.kernel, mesh=mesh,
        out_type=jax.ShapeDtypeStruct((N, O), jnp.float32),
        scratch_types=[
            pltpu.VMEM((NCHK, IC), jnp.int32),
            pltpu.VMEM((IC, O), jnp.float32),
            pltpu.VMEM((IC, O), jnp.float32),
            pltpu.SemaphoreType.DMA,
            pltpu.SemaphoreType.DMA,
        ],
    )
    def k(os_hbm, dest_hbm, out_hbm, dest_v, rows0, rows1, gsem0, gsem1):
        wid = lax.axis_index("s") * NC + lax.axis_index("c")
        base = wid * CH
        rows = [rows0, rows1]
        gsem = [gsem0, gsem1]
        pltpu.sync_copy(dest_hbm.at[wid], dest_v)
        gd = [None, None]
        gd[0] = pltpu.async_copy(os_hbm.at[dest_v.at[0]], rows0, gsem0)
        for kk in range(NCHK):
            nxt = kk + 1
            if nxt < NCHK:
                gd[nxt % 2] = pltpu.async_copy(
                    os_hbm.at[dest_v.at[nxt]], rows[nxt % 2], gsem[nxt % 2])
            gd[kk % 2].wait()
            pltpu.sync_copy(rows[kk % 2], out_hbm.at[pl.ds(base + kk * IC, IC)])

    return k(os_sorted, dest)


def kernel(x, W1, b1, W2, b2, Wr1, br1, Wr2, br2):
    chosen3, ranks3, counts2 = _router(x, Wr1, br1, Wr2, br2)
    counts = counts2[0]

    padded = ((counts + B - 1) // B) * B
    pad_off = (jnp.cumsum(padded) - padded).astype(jnp.int32)
    starts = jnp.arange(NBLK, dtype=jnp.int32) * B
    ends = pad_off + padded
    emap_raw = jnp.sum((starts[:, None] >= ends[None, :]).astype(jnp.int32), axis=1)
    emap = jnp.minimum(emap_raw, E - 1).astype(jnp.int32)
    act = ((emap_raw < E) & (starts - pad_off[emap] < counts[emap])).astype(jnp.int32)
    first = jnp.concatenate([jnp.ones((1,), jnp.int32),
                             (emap[1:] != emap[:-1]).astype(jnp.int32)])

    dest3 = _dest(pad_off, chosen3, ranks3)
    dest = dest3.reshape(NW, NCHK, IC)
    xs = _sc_scatter(x, dest)
    os_sorted = _grouped(emap, act, first, xs, W1, b1, W2, b2)
    return _sc_gather(os_sorted, dest)


# speedup vs baseline: 2.1147x; 1.1158x over previous
import functools

import jax
import jax.numpy as jnp
from jax import lax
from jax.experimental import pallas as pl
from jax.experimental.pallas import tpu as pltpu
from jax.experimental.pallas import tpu_sc as plsc

N, D, H, O, E, RH = 8192, 768, 768, 768, 8, 256
TB = 1024
NB_R = N // TB
B = 256
NBLK = N // B + E
P = NBLK * B
NC, NS = 2, 16
NW = NC * NS
CH = N // NW
IC = 64
NCHK = CH // IC

_HI = lax.Precision.HIGHEST


def _router_body(x_ref, wr1_ref, br1_ref, wr2_ref, br2_ref,
                 chosen_ref, ranks_ref, counts_ref, carry_ref):
    t = pl.program_id(0)

    @pl.when(t == 0)
    def _():
        carry_ref[...] = jnp.zeros_like(carry_ref)

    h = jnp.dot(x_ref[...], wr1_ref[...]) + br1_ref[...]
    logitsT = lax.dot_general(wr2_ref[...], h,
                              (((0,), (1,)), ((), ()))) + br2_ref[...]
    mx = jnp.max(logitsT, axis=0, keepdims=True)
    eidx = lax.broadcasted_iota(jnp.int32, (E, TB), 0)
    chosenT = jnp.min(jnp.where(logitsT >= mx, eidx, E), axis=0,
                      keepdims=True)
    onehotT = (eidx == chosenT)
    ohf = onehotT.astype(jnp.float32)
    ri = lax.broadcasted_iota(jnp.int32, (128, 128), 0)
    ci = lax.broadcasted_iota(jnp.int32, (128, 128), 1)
    triu = (ri < ci).astype(jnp.float32)
    carry = carry_ref[...]
    parts = []
    for g in range(TB // 128):
        blk = ohf[:, g * 128:(g + 1) * 128]
        local = jnp.dot(blk, triu)
        parts.append(local + carry)
        carry = carry + jnp.sum(blk, axis=1, keepdims=True)
    ranksT = jnp.concatenate(parts, axis=1)
    rank_tok = jnp.sum(jnp.where(onehotT, ranksT, 0.0), axis=0,
                       keepdims=True)
    chosen_ref[0] = chosenT
    ranks_ref[0] = rank_tok.astype(jnp.int32)
    counts_ref[...] = carry.astype(jnp.int32)
    carry_ref[...] = carry


def _router(x, Wr1, br1, Wr2, br2):
    return pl.pallas_call(
        _router_body,
        grid=(NB_R,),
        in_specs=[
            pl.BlockSpec((TB, D), lambda i: (i, 0)),
            pl.BlockSpec((D, RH), lambda i: (0, 0)),
            pl.BlockSpec((1, RH), lambda i: (0, 0)),
            pl.BlockSpec((RH, E), lambda i: (0, 0)),
            pl.BlockSpec((E, 1), lambda i: (0, 0)),
        ],
        out_specs=[
            pl.BlockSpec((1, 1, TB), lambda i: (i, 0, 0)),
            pl.BlockSpec((1, 1, TB), lambda i: (i, 0, 0)),
            pl.BlockSpec((E, 1), lambda i: (0, 0)),
        ],
        out_shape=[
            jax.ShapeDtypeStruct((NB_R, 1, TB), jnp.int32),
            jax.ShapeDtypeStruct((NB_R, 1, TB), jnp.int32),
            jax.ShapeDtypeStruct((E, 1), jnp.int32),
        ],
        scratch_shapes=[pltpu.VMEM((E, 1), jnp.float32)],
    )(x, Wr1, br1[None, :], Wr2, br2[:, None])


def _dest_body(counts_ref, chosen_ref, ranks_ref,
               dest_ref, emap_ref, act_ref, first_ref):
    counts_i = counts_ref[...]
    counts = counts_i.astype(jnp.float32)
    padded = (((counts_i + (B - 1)) // B) * B).astype(jnp.float32)
    te_r = lax.broadcasted_iota(jnp.int32, (E, E), 0)
    te_c = lax.broadcasted_iota(jnp.int32, (E, E), 1)
    trilE = (te_c < te_r).astype(jnp.float32)
    pad_off = jnp.dot(trilE, padded)
    ends = pad_off + padded
    starts = (lax.broadcasted_iota(jnp.int32, (1, NBLK), 1)
              * B).astype(jnp.float32)
    ge = (starts >= ends).astype(jnp.float32)
    emap_raw = jnp.sum(ge, axis=0, keepdims=True)
    emap = jnp.minimum(emap_raw, float(E - 1))
    esel = (lax.broadcasted_iota(jnp.int32, (E, NBLK), 0)
            == emap.astype(jnp.int32))
    pad_off_sel = jnp.sum(jnp.where(esel, pad_off, 0.0), axis=0,
                          keepdims=True)
    counts_sel = jnp.sum(jnp.where(esel, counts, 0.0), axis=0,
                         keepdims=True)
    off_b = starts - pad_off_sel
    act = (emap_raw < E) & (off_b < counts_sel)
    first = act & (off_b == 0.0)
    emap_ref[...] = emap.astype(jnp.int32)
    act_ref[...] = act.astype(jnp.int32)
    first_ref[...] = first.astype(jnp.int32)

    ch = chosen_ref[...]
    off = jnp.zeros_like(ch, dtype=jnp.float32)
    for e in range(E):
        off = jnp.where(ch == e, pad_off[e, 0], off)
    dest_ref[...] = off.astype(jnp.int32) + ranks_ref[...]


def _dest(counts2, chosen3, ranks3):
    return pl.pallas_call(
        _dest_body,
        grid=(1,),
        in_specs=[
            pl.BlockSpec((E, 1), lambda i: (0, 0)),
            pl.BlockSpec((NB_R, 1, TB), lambda i: (0, 0, 0)),
            pl.BlockSpec((NB_R, 1, TB), lambda i: (0, 0, 0)),
        ],
        out_specs=[
            pl.BlockSpec((NB_R, 1, TB), lambda i: (0, 0, 0)),
            pl.BlockSpec((1, NBLK), lambda i: (0, 0)),
            pl.BlockSpec((1, NBLK), lambda i: (0, 0)),
            pl.BlockSpec((1, NBLK), lambda i: (0, 0)),
        ],
        out_shape=[
            jax.ShapeDtypeStruct((NB_R, 1, TB), jnp.int32),
            jax.ShapeDtypeStruct((1, NBLK), jnp.int32),
            jax.ShapeDtypeStruct((1, NBLK), jnp.int32),
            jax.ShapeDtypeStruct((1, NBLK), jnp.int32),
        ],
    )(counts2, chosen3, ranks3)


def _sc_scatter(x, dest):
    mesh = plsc.VectorSubcoreMesh(core_axis_name="c", subcore_axis_name="s")

    @functools.partial(
        pl.kernel, mesh=mesh,
        out_type=jax.ShapeDtypeStruct((P, D), jnp.float32),
        scratch_types=[
            pltpu.VMEM((NCHK, IC), jnp.int32),
            pltpu.VMEM((IC, D), jnp.float32),
            pltpu.VMEM((IC, D), jnp.float32),
            pltpu.SemaphoreType.DMA,
            pltpu.SemaphoreType.DMA,
            pltpu.SemaphoreType.DMA,
        ],
    )
    def k(x_hbm, dest_hbm, xs_hbm, dest_v, rows0, rows1, lsem0, lsem1, ssem):
        wid = lax.axis_index("s") * NC + lax.axis_index("c")
        base = wid * CH
        rows = [rows0, rows1]
        lsem = [lsem0, lsem1]
        pltpu.sync_copy(dest_hbm.at[wid], dest_v)
        ld = [None, None]
        ld[0] = pltpu.async_copy(x_hbm.at[pl.ds(base, IC)], rows0, lsem0)
        for kk in range(NCHK):
            nxt = kk + 1
            if nxt < NCHK:
                ld[nxt % 2] = pltpu.async_copy(
                    x_hbm.at[pl.ds(base + nxt * IC, IC)], rows[nxt % 2],
                    lsem[nxt % 2])
            ld[kk % 2].wait()
            pltpu.async_copy(rows[kk % 2], xs_hbm.at[dest_v.at[kk]],
                             ssem).wait()

    return k(x, dest)


def _grouped_body(emap_ref, act_ref, first_ref, x_ref, w1_ref, b1_ref,
                  w2_ref, b2_ref, o_ref, m_v, c_v):
    b = pl.program_id(0)

    @pl.when((first_ref[b] != 0) & (act_ref[b] != 0))
    def _():
        w2 = w2_ref[0]
        m_v[...] = jnp.dot(w1_ref[0], w2)
        c_v[...] = jnp.dot(b1_ref[0], w2) + b2_ref[0]

    @pl.when(act_ref[b] != 0)
    def _():
        o_ref[...] = jnp.dot(x_ref[...], m_v[...]) + c_v[...]


def _grouped(emap, act, first, xs, W1, b1, W2, b2):
    spec = pltpu.PrefetchScalarGridSpec(
        num_scalar_prefetch=3,
        grid=(NBLK,),
        in_specs=[
            pl.BlockSpec((B, D), lambda b, em, ac, fi: (b, 0)),
            pl.BlockSpec((1, D, H), lambda b, em, ac, fi: (em[b], 0, 0)),
            pl.BlockSpec((1, 1, H), lambda b, em, ac, fi: (em[b], 0, 0)),
            pl.BlockSpec((1, H, O), lambda b, em, ac, fi: (em[b], 0, 0)),
            pl.BlockSpec((1, 1, O), lambda b, em, ac, fi: (em[b], 0, 0)),
        ],
        out_specs=pl.BlockSpec((B, O), lambda b, em, ac, fi: (b, 0)),
        scratch_shapes=[
            pltpu.VMEM((D, O), jnp.float32),
            pltpu.VMEM((1, O), jnp.float32),
        ],
    )
    return pl.pallas_call(
        _grouped_body,
        grid_spec=spec,
        out_shape=jax.ShapeDtypeStruct((P, O), jnp.float32),
    )(emap, act, first, xs, W1, b1[:, None, :], W2, b2[:, None, :])


def _sc_gather(os_sorted, dest):
    mesh = plsc.VectorSubcoreMesh(core_axis_name="c", subcore_axis_name="s")

    @functools.partial(
        pl.kernel, mesh=mesh,
        out_type=jax.ShapeDtypeStruct((N, O), jnp.float32),
        scratch_types=[
            pltpu.VMEM((NCHK, IC), jnp.int32),
            pltpu.VMEM((IC, O), jnp.float32),
            pltpu.VMEM((IC, O), jnp.float32),
            pltpu.SemaphoreType.DMA,
            pltpu.SemaphoreType.DMA,
        ],
    )
    def k(os_hbm, dest_hbm, out_hbm, dest_v, rows0, rows1, gsem0, gsem1):
        wid = lax.axis_index("s") * NC + lax.axis_index("c")
        base = wid * CH
        rows = [rows0, rows1]
        gsem = [gsem0, gsem1]
        pltpu.sync_copy(dest_hbm.at[wid], dest_v)
        gd = [None, None]
        gd[0] = pltpu.async_copy(os_hbm.at[dest_v.at[0]], rows0, gsem0)
        for kk in range(NCHK):
            nxt = kk + 1
            if nxt < NCHK:
                gd[nxt % 2] = pltpu.async_copy(
                    os_hbm.at[dest_v.at[nxt]], rows[nxt % 2], gsem[nxt % 2])
            gd[kk % 2].wait()
            pltpu.sync_copy(rows[kk % 2], out_hbm.at[pl.ds(base + kk * IC, IC)])

    return k(os_sorted, dest)


def kernel(x, W1, b1, W2, b2, Wr1, br1, Wr2, br2):
    chosen3, ranks3, counts2 = _router(x, Wr1, br1, Wr2, br2)
    dest3, emap2, act2, first2 = _dest(counts2, chosen3, ranks3)
    dest = dest3.reshape(NW, NCHK, IC)
    xs = _sc_scatter(x, dest)
    os_sorted = _grouped(emap2.reshape(NBLK), act2.reshape(NBLK),
                         first2.reshape(NBLK), xs, W1, b1, W2, b2)
    return _sc_gather(os_sorted, dest)


# speedup vs baseline: 2.2086x; 1.0444x over previous
import functools

import jax
import jax.numpy as jnp
from jax import lax
from jax.experimental import pallas as pl
from jax.experimental.pallas import tpu as pltpu
from jax.experimental.pallas import tpu_sc as plsc

N, D, H, O, E, RH = 8192, 768, 768, 768, 8, 256
TB = 1024
NB_R = N // TB
B = 256
NBLK = N // B + E
P = NBLK * B
NC, NS = 2, 16
NW = NC * NS
CH = N // NW
IC = 64
NCHK = CH // IC

_HI = lax.Precision.HIGHEST


D2 = D // 2


def _pack_bf16(xf):
    xb = xf.astype(jnp.bfloat16)
    lo = lax.bitcast_convert_type(xb[:, :D2], jnp.uint16).astype(jnp.uint32)
    hi = lax.bitcast_convert_type(xb[:, D2:], jnp.uint16).astype(jnp.uint32)
    return lax.bitcast_convert_type(lo | (hi << 16), jnp.int32)


def _unpack_bf16(xi):
    xu = lax.bitcast_convert_type(xi, jnp.uint32)
    lo = lax.bitcast_convert_type((xu & 0xFFFF).astype(jnp.uint16),
                                  jnp.bfloat16)
    hi = lax.bitcast_convert_type((xu >> 16).astype(jnp.uint16),
                                  jnp.bfloat16)
    return jnp.concatenate([lo, hi], axis=1)


def _router_body(x_ref, wr1_ref, br1_ref, wr2_ref, br2_ref,
                 chosen_ref, ranks_ref, counts_ref, xb_ref, carry_ref):
    t = pl.program_id(0)

    @pl.when(t == 0)
    def _():
        carry_ref[...] = jnp.zeros_like(carry_ref)

    h = jnp.dot(x_ref[...], wr1_ref[...]) + br1_ref[...]
    logitsT = lax.dot_general(wr2_ref[...], h,
                              (((0,), (1,)), ((), ()))) + br2_ref[...]
    mx = jnp.max(logitsT, axis=0, keepdims=True)
    eidx = lax.broadcasted_iota(jnp.int32, (E, TB), 0)
    chosenT = jnp.min(jnp.where(logitsT >= mx, eidx, E), axis=0,
                      keepdims=True)
    onehotT = (eidx == chosenT)
    ohf = onehotT.astype(jnp.float32)
    ri = lax.broadcasted_iota(jnp.int32, (128, 128), 0)
    ci = lax.broadcasted_iota(jnp.int32, (128, 128), 1)
    triu = (ri < ci).astype(jnp.float32)
    carry = carry_ref[...]
    parts = []
    for g in range(TB // 128):
        blk = ohf[:, g * 128:(g + 1) * 128]
        local = jnp.dot(blk, triu)
        parts.append(local + carry)
        carry = carry + jnp.sum(blk, axis=1, keepdims=True)
    ranksT = jnp.concatenate(parts, axis=1)
    rank_tok = jnp.sum(jnp.where(onehotT, ranksT, 0.0), axis=0,
                       keepdims=True)
    chosen_ref[0] = chosenT
    ranks_ref[0] = rank_tok.astype(jnp.int32)
    counts_ref[...] = carry.astype(jnp.int32)
    carry_ref[...] = carry
    xb_ref[...] = _pack_bf16(x_ref[...])


def _router(x, Wr1, br1, Wr2, br2):
    return pl.pallas_call(
        _router_body,
        grid=(NB_R,),
        in_specs=[
            pl.BlockSpec((TB, D), lambda i: (i, 0)),
            pl.BlockSpec((D, RH), lambda i: (0, 0)),
            pl.BlockSpec((1, RH), lambda i: (0, 0)),
            pl.BlockSpec((RH, E), lambda i: (0, 0)),
            pl.BlockSpec((E, 1), lambda i: (0, 0)),
        ],
        out_specs=[
            pl.BlockSpec((1, 1, TB), lambda i: (i, 0, 0)),
            pl.BlockSpec((1, 1, TB), lambda i: (i, 0, 0)),
            pl.BlockSpec((E, 1), lambda i: (0, 0)),
            pl.BlockSpec((TB, D2), lambda i: (i, 0)),
        ],
        out_shape=[
            jax.ShapeDtypeStruct((NB_R, 1, TB), jnp.int32),
            jax.ShapeDtypeStruct((NB_R, 1, TB), jnp.int32),
            jax.ShapeDtypeStruct((E, 1), jnp.int32),
            jax.ShapeDtypeStruct((N, D2), jnp.int32),
        ],
        scratch_shapes=[pltpu.VMEM((E, 1), jnp.float32)],
    )(x, Wr1, br1[None, :], Wr2, br2[:, None])


def _dest_body(counts_ref, chosen_ref, ranks_ref,
               dest_ref, emap_ref, act_ref, first_ref):
    counts_i = counts_ref[...]
    counts = counts_i.astype(jnp.float32)
    padded = (((counts_i + (B - 1)) // B) * B).astype(jnp.float32)
    te_r = lax.broadcasted_iota(jnp.int32, (E, E), 0)
    te_c = lax.broadcasted_iota(jnp.int32, (E, E), 1)
    trilE = (te_c < te_r).astype(jnp.float32)
    pad_off = jnp.dot(trilE, padded)
    ends = pad_off + padded
    starts = (lax.broadcasted_iota(jnp.int32, (1, NBLK), 1)
              * B).astype(jnp.float32)
    ge = (starts >= ends).astype(jnp.float32)
    emap_raw = jnp.sum(ge, axis=0, keepdims=True)
    emap = jnp.minimum(emap_raw, float(E - 1))
    esel = (lax.broadcasted_iota(jnp.int32, (E, NBLK), 0)
            == emap.astype(jnp.int32))
    pad_off_sel = jnp.sum(jnp.where(esel, pad_off, 0.0), axis=0,
                          keepdims=True)
    counts_sel = jnp.sum(jnp.where(esel, counts, 0.0), axis=0,
                         keepdims=True)
    off_b = starts - pad_off_sel
    act = (emap_raw < E) & (off_b < counts_sel)
    first = act & (off_b == 0.0)
    emap_ref[...] = emap.astype(jnp.int32)
    act_ref[...] = act.astype(jnp.int32)
    first_ref[...] = first.astype(jnp.int32)

    ch = chosen_ref[...]
    off = jnp.zeros_like(ch, dtype=jnp.float32)
    for e in range(E):
        off = jnp.where(ch == e, pad_off[e, 0], off)
    dest_ref[...] = off.astype(jnp.int32) + ranks_ref[...]


def _dest(counts2, chosen3, ranks3):
    return pl.pallas_call(
        _dest_body,
        grid=(1,),
        in_specs=[
            pl.BlockSpec((E, 1), lambda i: (0, 0)),
            pl.BlockSpec((NB_R, 1, TB), lambda i: (0, 0, 0)),
            pl.BlockSpec((NB_R, 1, TB), lambda i: (0, 0, 0)),
        ],
        out_specs=[
            pl.BlockSpec((NB_R, 1, TB), lambda i: (0, 0, 0)),
            pl.BlockSpec((1, NBLK), lambda i: (0, 0)),
            pl.BlockSpec((1, NBLK), lambda i: (0, 0)),
            pl.BlockSpec((1, NBLK), lambda i: (0, 0)),
        ],
        out_shape=[
            jax.ShapeDtypeStruct((NB_R, 1, TB), jnp.int32),
            jax.ShapeDtypeStruct((1, NBLK), jnp.int32),
            jax.ShapeDtypeStruct((1, NBLK), jnp.int32),
            jax.ShapeDtypeStruct((1, NBLK), jnp.int32),
        ],
    )(counts2, chosen3, ranks3)


def _sc_scatter(x, dest):
    mesh = plsc.VectorSubcoreMesh(core_axis_name="c", subcore_axis_name="s")

    @functools.partial(
        pl.kernel, mesh=mesh,
        out_type=jax.ShapeDtypeStruct((P, D2), jnp.int32),
        scratch_types=[
            pltpu.VMEM((NCHK, IC), jnp.int32),
            pltpu.VMEM((IC, D2), jnp.int32),
            pltpu.VMEM((IC, D2), jnp.int32),
            pltpu.SemaphoreType.DMA,
            pltpu.SemaphoreType.DMA,
            pltpu.SemaphoreType.DMA,
        ],
    )
    def k(x_hbm, dest_hbm, xs_hbm, dest_v, rows0, rows1, lsem0, lsem1, ssem):
        wid = lax.axis_index("s") * NC + lax.axis_index("c")
        base = wid * CH
        rows = [rows0, rows1]
        lsem = [lsem0, lsem1]
        pltpu.sync_copy(dest_hbm.at[wid], dest_v)
        ld = [None, None]
        ld[0] = pltpu.async_copy(x_hbm.at[pl.ds(base, IC)], rows0, lsem0)
        for kk in range(NCHK):
            nxt = kk + 1
            if nxt < NCHK:
                ld[nxt % 2] = pltpu.async_copy(
                    x_hbm.at[pl.ds(base + nxt * IC, IC)], rows[nxt % 2],
                    lsem[nxt % 2])
            ld[kk % 2].wait()
            pltpu.async_copy(rows[kk % 2], xs_hbm.at[dest_v.at[kk]],
                             ssem).wait()

    return k(x, dest)


def _grouped_body(emap_ref, act_ref, first_ref, x_ref, w1_ref, b1_ref,
                  w2_ref, b2_ref, o_ref, m_v, c_v):
    b = pl.program_id(0)

    @pl.when((first_ref[b] != 0) & (act_ref[b] != 0))
    def _():
        w2 = w2_ref[0]
        m = jnp.dot(w1_ref[0].astype(jnp.bfloat16), w2.astype(jnp.bfloat16),
                    preferred_element_type=jnp.float32)
        m_v[...] = m.astype(jnp.bfloat16)
        c_v[...] = jnp.dot(b1_ref[0], w2) + b2_ref[0]

    @pl.when(act_ref[b] != 0)
    def _():
        xb = _unpack_bf16(x_ref[...])
        o_ref[...] = jnp.dot(xb, m_v[...],
                             preferred_element_type=jnp.float32) + c_v[...]


def _grouped(emap, act, first, xs, W1, b1, W2, b2):
    spec = pltpu.PrefetchScalarGridSpec(
        num_scalar_prefetch=3,
        grid=(NBLK,),
        in_specs=[
            pl.BlockSpec((B, D2), lambda b, em, ac, fi: (b, 0)),
            pl.BlockSpec((1, D, H), lambda b, em, ac, fi: (em[b], 0, 0)),
            pl.BlockSpec((1, 1, H), lambda b, em, ac, fi: (em[b], 0, 0)),
            pl.BlockSpec((1, H, O), lambda b, em, ac, fi: (em[b], 0, 0)),
            pl.BlockSpec((1, 1, O), lambda b, em, ac, fi: (em[b], 0, 0)),
        ],
        out_specs=pl.BlockSpec((B, O), lambda b, em, ac, fi: (b, 0)),
        scratch_shapes=[
            pltpu.VMEM((D, O), jnp.bfloat16),
            pltpu.VMEM((1, O), jnp.float32),
        ],
    )
    return pl.pallas_call(
        _grouped_body,
        grid_spec=spec,
        out_shape=jax.ShapeDtypeStruct((P, O), jnp.float32),
    )(emap, act, first, xs, W1, b1[:, None, :], W2, b2[:, None, :])


def _sc_gather(os_sorted, dest):
    mesh = plsc.VectorSubcoreMesh(core_axis_name="c", subcore_axis_name="s")

    @functools.partial(
        pl.kernel, mesh=mesh,
        out_type=jax.ShapeDtypeStruct((N, O), jnp.float32),
        scratch_types=[
            pltpu.VMEM((NCHK, IC), jnp.int32),
            pltpu.VMEM((IC, O), jnp.float32),
            pltpu.VMEM((IC, O), jnp.float32),
            pltpu.SemaphoreType.DMA,
            pltpu.SemaphoreType.DMA,
        ],
    )
    def k(os_hbm, dest_hbm, out_hbm, dest_v, rows0, rows1, gsem0, gsem1):
        wid = lax.axis_index("s") * NC + lax.axis_index("c")
        base = wid * CH
        rows = [rows0, rows1]
        gsem = [gsem0, gsem1]
        pltpu.sync_copy(dest_hbm.at[wid], dest_v)
        gd = [None, None]
        gd[0] = pltpu.async_copy(os_hbm.at[dest_v.at[0]], rows0, gsem0)
        for kk in range(NCHK):
            nxt = kk + 1
            if nxt < NCHK:
                gd[nxt % 2] = pltpu.async_copy(
                    os_hbm.at[dest_v.at[nxt]], rows[nxt % 2], gsem[nxt % 2])
            gd[kk % 2].wait()
            pltpu.sync_copy(rows[kk % 2], out_hbm.at[pl.ds(base + kk * IC, IC)])

    return k(os_sorted, dest)


def kernel(x, W1, b1, W2, b2, Wr1, br1, Wr2, br2):
    chosen3, ranks3, counts2, xb = _router(x, Wr1, br1, Wr2, br2)
    dest3, emap2, act2, first2 = _dest(counts2, chosen3, ranks3)
    dest = dest3.reshape(NW, NCHK, IC)
    xs = _sc_scatter(xb, dest)
    os_sorted = _grouped(emap2.reshape(NBLK), act2.reshape(NBLK),
                         first2.reshape(NBLK), xs, W1, b1, W2, b2)
    return _sc_gather(os_sorted, dest)


# speedup vs baseline: 2.2926x; 1.0380x over previous
import functools

import jax
import jax.numpy as jnp
from jax import lax
from jax.experimental import pallas as pl
from jax.experimental.pallas import tpu as pltpu
from jax.experimental.pallas import tpu_sc as plsc

N, D, H, O, E, RH = 8192, 768, 768, 768, 8, 256
TB = 1024
NB_R = N // TB
B = 512
NBLK = N // B + E
P = NBLK * B
NC, NS = 2, 16
NW = NC * NS
CH = N // NW
IC = 64
NCHK = CH // IC

_HI = lax.Precision.HIGHEST


D2 = D // 2


def _pack_bf16(xf):
    xb = xf.astype(jnp.bfloat16)
    lo = lax.bitcast_convert_type(xb[:, :D2], jnp.uint16).astype(jnp.uint32)
    hi = lax.bitcast_convert_type(xb[:, D2:], jnp.uint16).astype(jnp.uint32)
    return lax.bitcast_convert_type(lo | (hi << 16), jnp.int32)


def _unpack_bf16(xi):
    xu = lax.bitcast_convert_type(xi, jnp.uint32)
    lo = lax.bitcast_convert_type((xu & 0xFFFF).astype(jnp.uint16),
                                  jnp.bfloat16)
    hi = lax.bitcast_convert_type((xu >> 16).astype(jnp.uint16),
                                  jnp.bfloat16)
    return jnp.concatenate([lo, hi], axis=1)


def _router_body(x_ref, wr1_ref, br1_ref, wr2_ref, br2_ref,
                 chosen_ref, ranks_ref, counts_ref, xb_ref, carry_ref):
    t = pl.program_id(0)

    @pl.when(t == 0)
    def _():
        carry_ref[...] = jnp.zeros_like(carry_ref)

    h = jnp.dot(x_ref[...], wr1_ref[...]) + br1_ref[...]
    logitsT = lax.dot_general(wr2_ref[...], h,
                              (((0,), (1,)), ((), ()))) + br2_ref[...]
    mx = jnp.max(logitsT, axis=0, keepdims=True)
    eidx = lax.broadcasted_iota(jnp.int32, (E, TB), 0)
    chosenT = jnp.min(jnp.where(logitsT >= mx, eidx, E), axis=0,
                      keepdims=True)
    onehotT = (eidx == chosenT)
    ohf = onehotT.astype(jnp.float32)
    ri = lax.broadcasted_iota(jnp.int32, (128, 128), 0)
    ci = lax.broadcasted_iota(jnp.int32, (128, 128), 1)
    triu = (ri < ci).astype(jnp.float32)
    carry = carry_ref[...]
    parts = []
    for g in range(TB // 128):
        blk = ohf[:, g * 128:(g + 1) * 128]
        local = jnp.dot(blk, triu)
        parts.append(local + carry)
        carry = carry + jnp.sum(blk, axis=1, keepdims=True)
    ranksT = jnp.concatenate(parts, axis=1)
    rank_tok = jnp.sum(jnp.where(onehotT, ranksT, 0.0), axis=0,
                       keepdims=True)
    chosen_ref[0] = chosenT
    ranks_ref[0] = rank_tok.astype(jnp.int32)
    counts_ref[...] = carry.astype(jnp.int32)
    carry_ref[...] = carry
    xb_ref[...] = _pack_bf16(x_ref[...])


def _router(x, Wr1, br1, Wr2, br2):
    return pl.pallas_call(
        _router_body,
        grid=(NB_R,),
        in_specs=[
            pl.BlockSpec((TB, D), lambda i: (i, 0)),
            pl.BlockSpec((D, RH), lambda i: (0, 0)),
            pl.BlockSpec((1, RH), lambda i: (0, 0)),
            pl.BlockSpec((RH, E), lambda i: (0, 0)),
            pl.BlockSpec((E, 1), lambda i: (0, 0)),
        ],
        out_specs=[
            pl.BlockSpec((1, 1, TB), lambda i: (i, 0, 0)),
            pl.BlockSpec((1, 1, TB), lambda i: (i, 0, 0)),
            pl.BlockSpec((E, 1), lambda i: (0, 0)),
            pl.BlockSpec((TB, D2), lambda i: (i, 0)),
        ],
        out_shape=[
            jax.ShapeDtypeStruct((NB_R, 1, TB), jnp.int32),
            jax.ShapeDtypeStruct((NB_R, 1, TB), jnp.int32),
            jax.ShapeDtypeStruct((E, 1), jnp.int32),
            jax.ShapeDtypeStruct((N, D2), jnp.int32),
        ],
        scratch_shapes=[pltpu.VMEM((E, 1), jnp.float32)],
    )(x, Wr1, br1[None, :], Wr2, br2[:, None])


def _dest_body(counts_ref, chosen_ref, ranks_ref,
               dest_ref, emap_ref, act_ref, first_ref):
    counts_i = counts_ref[...]
    counts = counts_i.astype(jnp.float32)
    padded = (((counts_i + (B - 1)) // B) * B).astype(jnp.float32)
    te_r = lax.broadcasted_iota(jnp.int32, (E, E), 0)
    te_c = lax.broadcasted_iota(jnp.int32, (E, E), 1)
    trilE = (te_c < te_r).astype(jnp.float32)
    pad_off = jnp.dot(trilE, padded)
    ends = pad_off + padded
    starts = (lax.broadcasted_iota(jnp.int32, (1, NBLK), 1)
              * B).astype(jnp.float32)
    ge = (starts >= ends).astype(jnp.float32)
    emap_raw = jnp.sum(ge, axis=0, keepdims=True)
    emap = jnp.minimum(emap_raw, float(E - 1))
    esel = (lax.broadcasted_iota(jnp.int32, (E, NBLK), 0)
            == emap.astype(jnp.int32))
    pad_off_sel = jnp.sum(jnp.where(esel, pad_off, 0.0), axis=0,
                          keepdims=True)
    counts_sel = jnp.sum(jnp.where(esel, counts, 0.0), axis=0,
                         keepdims=True)
    off_b = starts - pad_off_sel
    act = (emap_raw < E) & (off_b < counts_sel)
    first = act & (off_b == 0.0)
    emap_ref[...] = emap.astype(jnp.int32)
    act_ref[...] = act.astype(jnp.int32)
    first_ref[...] = first.astype(jnp.int32)

    ch = chosen_ref[...]
    off = jnp.zeros_like(ch, dtype=jnp.float32)
    for e in range(E):
        off = jnp.where(ch == e, pad_off[e, 0], off)
    dest_ref[...] = off.astype(jnp.int32) + ranks_ref[...]


def _dest(counts2, chosen3, ranks3):
    return pl.pallas_call(
        _dest_body,
        grid=(1,),
        in_specs=[
            pl.BlockSpec((E, 1), lambda i: (0, 0)),
            pl.BlockSpec((NB_R, 1, TB), lambda i: (0, 0, 0)),
            pl.BlockSpec((NB_R, 1, TB), lambda i: (0, 0, 0)),
        ],
        out_specs=[
            pl.BlockSpec((NB_R, 1, TB), lambda i: (0, 0, 0)),
            pl.BlockSpec((1, NBLK), lambda i: (0, 0)),
            pl.BlockSpec((1, NBLK), lambda i: (0, 0)),
            pl.BlockSpec((1, NBLK), lambda i: (0, 0)),
        ],
        out_shape=[
            jax.ShapeDtypeStruct((NB_R, 1, TB), jnp.int32),
            jax.ShapeDtypeStruct((1, NBLK), jnp.int32),
            jax.ShapeDtypeStruct((1, NBLK), jnp.int32),
            jax.ShapeDtypeStruct((1, NBLK), jnp.int32),
        ],
    )(counts2, chosen3, ranks3)


def _sc_scatter(x, dest):
    mesh = plsc.VectorSubcoreMesh(core_axis_name="c", subcore_axis_name="s")

    @functools.partial(
        pl.kernel, mesh=mesh,
        out_type=jax.ShapeDtypeStruct((P, D2), jnp.int32),
        scratch_types=[
            pltpu.VMEM((NCHK, IC), jnp.int32),
            pltpu.VMEM((IC, D2), jnp.int32),
            pltpu.VMEM((IC, D2), jnp.int32),
            pltpu.SemaphoreType.DMA,
            pltpu.SemaphoreType.DMA,
            pltpu.SemaphoreType.DMA,
        ],
    )
    def k(x_hbm, dest_hbm, xs_hbm, dest_v, rows0, rows1, lsem0, lsem1, ssem):
        wid = lax.axis_index("s") * NC + lax.axis_index("c")
        base = wid * CH
        rows = [rows0, rows1]
        lsem = [lsem0, lsem1]
        pltpu.sync_copy(dest_hbm.at[wid], dest_v)
        ld = [None, None]
        ld[0] = pltpu.async_copy(x_hbm.at[pl.ds(base, IC)], rows0, lsem0)
        for kk in range(NCHK):
            nxt = kk + 1
            if nxt < NCHK:
                ld[nxt % 2] = pltpu.async_copy(
                    x_hbm.at[pl.ds(base + nxt * IC, IC)], rows[nxt % 2],
                    lsem[nxt % 2])
            ld[kk % 2].wait()
            pltpu.async_copy(rows[kk % 2], xs_hbm.at[dest_v.at[kk]],
                             ssem).wait()

    return k(x, dest)


def _grouped_body(emap_ref, act_ref, first_ref, x_ref, w1_ref, b1_ref,
                  w2_ref, b2_ref, o_ref, m_v, c_v):
    b = pl.program_id(0)

    @pl.when((first_ref[b] != 0) & (act_ref[b] != 0))
    def _():
        w2 = w2_ref[0]
        m = jnp.dot(w1_ref[0].astype(jnp.bfloat16), w2.astype(jnp.bfloat16),
                    preferred_element_type=jnp.float32)
        m_v[...] = m.astype(jnp.bfloat16)
        c_v[...] = jnp.dot(b1_ref[0], w2) + b2_ref[0]

    @pl.when(act_ref[b] != 0)
    def _():
        xb = _unpack_bf16(x_ref[...])
        o_ref[...] = jnp.dot(xb, m_v[...],
                             preferred_element_type=jnp.float32) + c_v[...]


def _grouped(emap, act, first, xs, W1, b1, W2, b2):
    spec = pltpu.PrefetchScalarGridSpec(
        num_scalar_prefetch=3,
        grid=(NBLK,),
        in_specs=[
            pl.BlockSpec((B, D2), lambda b, em, ac, fi: (b, 0)),
            pl.BlockSpec((1, D, H), lambda b, em, ac, fi: (em[b], 0, 0)),
            pl.BlockSpec((1, 1, H), lambda b, em, ac, fi: (em[b], 0, 0)),
            pl.BlockSpec((1, H, O), lambda b, em, ac, fi: (em[b], 0, 0)),
            pl.BlockSpec((1, 1, O), lambda b, em, ac, fi: (em[b], 0, 0)),
        ],
        out_specs=pl.BlockSpec((B, O), lambda b, em, ac, fi: (b, 0)),
        scratch_shapes=[
            pltpu.VMEM((D, O), jnp.bfloat16),
            pltpu.VMEM((1, O), jnp.float32),
        ],
    )
    return pl.pallas_call(
        _grouped_body,
        grid_spec=spec,
        out_shape=jax.ShapeDtypeStruct((P, O), jnp.float32),
    )(emap, act, first, xs, W1, b1[:, None, :], W2, b2[:, None, :])


def _sc_gather(os_sorted, dest):
    mesh = plsc.VectorSubcoreMesh(core_axis_name="c", subcore_axis_name="s")

    @functools.partial(
        pl.kernel, mesh=mesh,
        out_type=jax.ShapeDtypeStruct((N, O), jnp.float32),
        scratch_types=[
            pltpu.VMEM((NCHK, IC), jnp.int32),
            pltpu.VMEM((IC, O), jnp.float32),
            pltpu.VMEM((IC, O), jnp.float32),
            pltpu.SemaphoreType.DMA,
            pltpu.SemaphoreType.DMA,
        ],
    )
    def k(os_hbm, dest_hbm, out_hbm, dest_v, rows0, rows1, gsem0, gsem1):
        wid = lax.axis_index("s") * NC + lax.axis_index("c")
        base = wid * CH
        rows = [rows0, rows1]
        gsem = [gsem0, gsem1]
        pltpu.sync_copy(dest_hbm.at[wid], dest_v)
        gd = [None, None]
        gd[0] = pltpu.async_copy(os_hbm.at[dest_v.at[0]], rows0, gsem0)
        for kk in range(NCHK):
            nxt = kk + 1
            if nxt < NCHK:
                gd[nxt % 2] = pltpu.async_copy(
                    os_hbm.at[dest_v.at[nxt]], rows[nxt % 2], gsem[nxt % 2])
            gd[kk % 2].wait()
            pltpu.sync_copy(rows[kk % 2], out_hbm.at[pl.ds(base + kk * IC, IC)])

    return k(os_sorted, dest)


def kernel(x, W1, b1, W2, b2, Wr1, br1, Wr2, br2):
    chosen3, ranks3, counts2, xb = _router(x, Wr1, br1, Wr2, br2)
    dest3, emap2, act2, first2 = _dest(counts2, chosen3, ranks3)
    dest = dest3.reshape(NW, NCHK, IC)
    xs = _sc_scatter(xb, dest)
    os_sorted = _grouped(emap2.reshape(NBLK), act2.reshape(NBLK),
                         first2.reshape(NBLK), xs, W1, b1, W2, b2)
    return _sc_gather(os_sorted, dest)
